# baseline (speedup 1.0000x reference)
import functools

import jax
import jax.numpy as jnp
from jax import lax
from jax.experimental import pallas as pl
from jax.experimental.pallas import tpu as pltpu
from jax.experimental.pallas import tpu_sc as plsc

B = 64
DIN = 256
H = 128
KDIM = 128
DICT = 100000
A2CH = 128
NA = 6

TILE = 10000
NT = DICT // TILE

_SC_NC = 2
_ROWS_PER_W = 8
_NW = B // _ROWS_PER_W

_S0CH = 400
_S0NCH = DICT // _S0CH
_S0W = 32
_S0K = (_S0NCH + _S0W - 1) // _S0W


def _k1_body(q_ref, idc_ref, idr_ref, keys_ref,
             nk_ref, best_ref,
             bv_scr, bi_scr, qp_scr, lm_scr):
    g = pl.program_id(0)

    k_tile = keys_ref[...]

    q = q_ref[...]
    qn = q / (jnp.sqrt(jnp.sum(q * q, axis=1, keepdims=True)) + 1e-8)
    kn = k_tile / (jnp.sqrt(jnp.sum(k_tile * k_tile, axis=1, keepdims=True)) + 1e-8)
    s = jax.lax.dot_general(qn, kn, (((1,), (1,)), ((), ())),
                            preferred_element_type=jnp.float32)

    lmax = jnp.max(s, axis=1, keepdims=True)
    li = jax.lax.broadcasted_iota(jnp.int32, s.shape, 1)
    lidx = jnp.min(jnp.where(s == lmax, li, TILE), axis=1, keepdims=True) + g * TILE

    @pl.when(g == 0)
    def _init():
        bv_scr[...] = jnp.full((B, 1), -jnp.inf, jnp.float32)
        bi_scr[...] = jnp.zeros((B, 1), jnp.int32)
        idc = idc_ref[...]
        idr = idr_ref[...]
        jj = jax.lax.broadcasted_iota(jnp.int32, (B, B), 1)
        eq = idc == idr
        last = jnp.max(jnp.where(eq, jj, -1), axis=1, keepdims=True)
        P = (jj == last).astype(jnp.float32)
        qp_scr[...] = jax.lax.dot_general(P, q, (((1,), (0,)), ((), ())),
                                          preferred_element_type=jnp.float32)
        ii = jax.lax.broadcasted_iota(jnp.int32, (B, B), 0)
        lastcol = jnp.max(jnp.where(eq, ii, -1), axis=0, keepdims=True)
        jb = jax.lax.broadcasted_iota(jnp.int32, (1, B), 1)
        lm_scr[...] = (lastcol == jb).astype(jnp.float32)

    upd = lmax > bv_scr[...]
    bv_scr[...] = jnp.where(upd, lmax, bv_scr[...])
    bi_scr[...] = jnp.where(upd, lidx, bi_scr[...])

    it = jax.lax.broadcasted_iota(jnp.int32, (TILE, 1), 0) + g * TILE
    oh = (it == idr_ref[...]).astype(jnp.float32) * lm_scr[...]
    hit = jnp.max(oh, axis=1, keepdims=True)
    repl = jax.lax.dot_general(oh.astype(jnp.bfloat16),
                               qp_scr[...].astype(jnp.bfloat16),
                               (((1,), (0,)), ((), ())),
                               preferred_element_type=jnp.float32)
    nk_ref[...] = k_tile * (1.0 - hit) + repl

    @pl.when(g == NT - 1)
    def _fin():
        best_ref[...] = bi_scr[...]


def _sc_wid():
    return lax.axis_index("s") * _SC_NC + lax.axis_index("c")


def _s0_body(vals_hbm, out_hbm, buf0, buf1, sem_in, sem_out):
    wid = _sc_wid()
    bufs = (buf0, buf1)

    def chunk(k):
        return wid + _S0W * k

    def in_cp(k):
        return pltpu.make_async_copy(
            vals_hbm.at[pl.ds(chunk(k) * _S0CH, _S0CH), :], bufs[k % 2], sem_in)

    def out_cp(k):
        return pltpu.make_async_copy(
            bufs[k % 2], out_hbm.at[pl.ds(chunk(k) * _S0CH, _S0CH), :], sem_out)

    for k in range(_S0K):
        if k >= 2:
            @pl.when(chunk(k - 2) < _S0NCH)
            def _(k=k):
                out_cp(k - 2).wait()

        @pl.when(chunk(k) < _S0NCH)
        def _(k=k):
            in_cp(k).start()

        if k >= 1:
            @pl.when(chunk(k - 1) < _S0NCH)
            def _(k=k):
                in_cp(k - 1).wait()
                out_cp(k - 1).start()

    @pl.when(chunk(_S0K - 1) < _S0NCH)
    def _():
        in_cp(_S0K - 1).wait()
        out_cp(_S0K - 1).start()

    if _S0K >= 2:
        @pl.when(chunk(_S0K - 2) < _S0NCH)
        def _():
            out_cp(_S0K - 2).wait()

    @pl.when(chunk(_S0K - 1) < _S0NCH)
    def _():
        out_cp(_S0K - 1).wait()


def _s1_body(best_hbm, vals_hbm, out_hbm, idx_v, rows_v, sem):
    wid = _sc_wid()

    @pl.when(wid < _NW)
    def _():
        base = wid * _ROWS_PER_W
        pltpu.sync_copy(best_hbm.at[pl.ds(base, _ROWS_PER_W)], idx_v)
        pltpu.async_copy(vals_hbm.at[idx_v], rows_v, sem).wait()
        pltpu.sync_copy(rows_v, out_hbm.at[pl.ds(base, _ROWS_PER_W)])


def _s2_body(ids_hbm, cp_hbm, nv_ref, idx_v, crows_v, sem):
    wid = _sc_wid()

    @pl.when(wid < _NW)
    def _():
        base = wid * _ROWS_PER_W
        pltpu.sync_copy(ids_hbm.at[pl.ds(base, _ROWS_PER_W)], idx_v)
        pltpu.sync_copy(cp_hbm.at[pl.ds(base, _ROWS_PER_W)], crows_v)
        pltpu.async_copy(crows_v, nv_ref.at[idx_v], sem).wait()


def _k2_body(best_ref, bid_ref, x_ref, h_ref, c_ref, Wi_ref, bi_ref,
             Wh_ref, bh_ref, W1_ref, b1_ref, Wa_ref, ba_ref, Wc_ref, bc_ref,
             idc_ref, idr_ref, vals_any,
             a_ref, prob_ref, v_ref, ent_ref, ht_ref, ct_ref, cp_ref,
             mem_ref, sem):
    def dotT(a, w):
        return jax.lax.dot_general(a, w, (((1,), (1,)), ((), ())),
                                   preferred_element_type=jnp.float32)

    for b in range(B):
        idx = best_ref[b]
        pltpu.make_async_copy(vals_any.at[pl.ds(idx, 1), :],
                              mem_ref.at[pl.ds(b, 1), :], sem).start()

    x = x_ref[...]
    hh = h_ref[...]
    preact = (dotT(x, Wi_ref[...]) + bi_ref[...]
              + dotT(hh, Wh_ref[...]) + bh_ref[...])
    gates = jax.nn.sigmoid(preact[:, :4 * H])
    f_t = gates[:, 0:H]
    i_t = gates[:, H:2 * H]
    o_t = gates[:, 2 * H:3 * H]
    r_t = gates[:, 3 * H:4 * H]
    c_new = jnp.tanh(preact[:, 4 * H:5 * H])
    c_t = f_t * c_ref[...] + i_t * c_new

    for b in range(B):
        idx = best_ref[b]
        pltpu.make_async_copy(vals_any.at[pl.ds(idx, 1), :],
                              mem_ref.at[pl.ds(b, 1), :], sem).wait()

    m_t = jnp.tanh(mem_ref[...])
    c_t = c_t + r_t * m_t
    h_t = o_t * jnp.tanh(c_t)
    ht_ref[...] = h_t
    ct_ref[...] = c_t

    hid = jax.nn.relu(dotT(c_t, W1_ref[...]) + b1_ref[...])
    logits = dotT(hid, Wa_ref[...]) + ba_ref[...]
    lm = jnp.max(logits, axis=1, keepdims=True)
    e = jnp.exp(logits - lm)
    pi = e / jnp.sum(e, axis=1, keepdims=True)
    ent_ref[...] = -jnp.sum(pi * jnp.log(pi + 1e-12), axis=1, keepdims=True)
    pmax = jnp.max(pi, axis=1, keepdims=True)
    ai = jax.lax.broadcasted_iota(jnp.int32, pi.shape, 1)
    a_ref[...] = jnp.min(jnp.where(pi == pmax, ai, NA), axis=1, keepdims=True)
    prob_ref[...] = jnp.log(pmax + 1e-12)
    v_ref[...] = jnp.sum(hid * Wc_ref[...], axis=1, keepdims=True) + bc_ref[0]

    idc = idc_ref[...]
    idr = idr_ref[...]
    jj = jax.lax.broadcasted_iota(jnp.int32, (B, B), 1)
    eq = idc == idr
    last = jnp.max(jnp.where(eq, jj, -1), axis=1, keepdims=True)
    P = (jj == last).astype(jnp.float32)
    cp_ref[...] = jax.lax.dot_general(P, c_t, (((1,), (0,)), ((), ())),
                                      preferred_element_type=jnp.float32)


def kernel(obs_bar_reward, barcode_tensor, barcode_id, h, c, Wi, bi, Wh, bh,
           dnd_keys, dnd_vals, W1, b1, Wa, ba, Wc, bc):
    f32 = jnp.float32
    idc = barcode_id.reshape(B, 1)
    idr = barcode_id.reshape(1, B)

    vmem = lambda shape: pl.BlockSpec(shape, lambda g: (0, 0))
    tile = lambda w: pl.BlockSpec((TILE, w), lambda g: (g, 0))

    sc_mesh = plsc.VectorSubcoreMesh(core_axis_name="c", subcore_axis_name="s")

    nv_pre = pl.kernel(
        _s0_body,
        out_type=jax.ShapeDtypeStruct((DICT, H), f32),
        mesh=sc_mesh,
        scratch_types=[pltpu.VMEM((_S0CH, H), f32),
                       pltpu.VMEM((_S0CH, H), f32),
                       pltpu.SemaphoreType.DMA,
                       pltpu.SemaphoreType.DMA],
        name="dnd_vals_copy",
        cost_estimate=pl.CostEstimate(flops=0, transcendentals=0,
                                      bytes_accessed=2 * DICT * H * 4),
    )(dnd_vals)

    new_keys, best_id = pl.pallas_call(
        _k1_body,
        grid=(NT,),
        in_specs=[vmem((B, KDIM)), vmem((B, 1)), vmem((1, B)),
                  tile(KDIM)],
        out_specs=[tile(KDIM), vmem((B, 1))],
        out_shape=[jax.ShapeDtypeStruct((DICT, KDIM), f32),
                   jax.ShapeDtypeStruct((B, 1), jnp.int32)],
        scratch_shapes=[pltpu.VMEM((B, 1), f32), pltpu.VMEM((B, 1), jnp.int32),
                        pltpu.VMEM((B, KDIM), f32), pltpu.VMEM((1, B), f32)],
    )(barcode_tensor, idc, idr, dnd_keys)

    smem1d = pl.BlockSpec(memory_space=pltpu.SMEM)
    vfull = pl.BlockSpec(memory_space=pltpu.VMEM)
    anyspec = pl.BlockSpec(memory_space=pl.ANY)

    (a_t, prob_a_t, v_t, entropy, h_t, c_t, c_payload) = pl.pallas_call(
        _k2_body,
        in_specs=[smem1d, smem1d, vfull, vfull, vfull, vfull, vfull,
                  vfull, vfull, vfull, vfull, vfull, vfull, vfull, smem1d,
                  vfull, vfull, anyspec],
        out_specs=[vfull, vfull, vfull, vfull, vfull, vfull, vfull],
        out_shape=[jax.ShapeDtypeStruct((B, 1), jnp.int32),
                   jax.ShapeDtypeStruct((B, 1), f32),
                   jax.ShapeDtypeStruct((B, 1), f32),
                   jax.ShapeDtypeStruct((B, 1), f32),
                   jax.ShapeDtypeStruct((B, H), f32),
                   jax.ShapeDtypeStruct((B, H), f32),
                   jax.ShapeDtypeStruct((B, H), f32)],
        scratch_shapes=[pltpu.VMEM((B, H), f32), pltpu.SemaphoreType.DMA],
    )(best_id.reshape(B), barcode_id, obs_bar_reward, h, c,
      Wi, bi.reshape(1, 5 * H), Wh, bh.reshape(1, 5 * H),
      W1, b1.reshape(1, A2CH), Wa, ba.reshape(1, NA), Wc, bc,
      idc, idr, dnd_vals)

    nv_ref = jax.new_ref(nv_pre)
    pl.kernel(
        _s2_body,
        out_type=(),
        mesh=sc_mesh,
        scratch_types=[pltpu.VMEM((_ROWS_PER_W,), jnp.int32),
                       pltpu.VMEM((_ROWS_PER_W, H), f32),
                       pltpu.SemaphoreType.DMA],
        name="dnd_scatter",
    )(barcode_id, c_payload, nv_ref)
    new_vals = nv_ref[...]

    return (a_t.reshape(B), prob_a_t.reshape(B), v_t, entropy.reshape(B),
            h_t, c_t, best_id.reshape(B), new_keys, new_vals)


# speedup vs baseline: 1.0348x; 1.0348x over previous
import functools

import jax
import jax.numpy as jnp
from jax import lax
from jax.experimental import pallas as pl
from jax.experimental.pallas import tpu as pltpu
from jax.experimental.pallas import tpu_sc as plsc

B = 64
DIN = 256
H = 128
KDIM = 128
DICT = 100000
A2CH = 128
NA = 6

TILE = 10000
NT = DICT // TILE

_SC_NC = 2
_ROWS_PER_W = 8
_NW = B // _ROWS_PER_W

_S0CH = 400
_S0NCH = DICT // _S0CH
_S0W = 32
_S0K = (_S0NCH + _S0W - 1) // _S0W


def _k1_body(q_ref, idc_ref, idr_ref, keys_ref,
             nk_ref, best_ref, qp_ref,
             bv_scr, bi_scr):
    g = pl.program_id(0)

    k_tile = keys_ref[...]
    nk_ref[...] = k_tile

    q = q_ref[...]
    qn = q / (jnp.sqrt(jnp.sum(q * q, axis=1, keepdims=True)) + 1e-8)
    kn = k_tile / (jnp.sqrt(jnp.sum(k_tile * k_tile, axis=1, keepdims=True)) + 1e-8)
    s = jax.lax.dot_general(qn, kn, (((1,), (1,)), ((), ())),
                            preferred_element_type=jnp.float32)

    lmax = jnp.max(s, axis=1, keepdims=True)
    li = jax.lax.broadcasted_iota(jnp.int32, s.shape, 1)
    lidx = jnp.min(jnp.where(s == lmax, li, TILE), axis=1, keepdims=True) + g * TILE

    @pl.when(g == 0)
    def _init():
        bv_scr[...] = jnp.full((B, 1), -jnp.inf, jnp.float32)
        bi_scr[...] = jnp.zeros((B, 1), jnp.int32)
        idc = idc_ref[...]
        idr = idr_ref[...]
        jj = jax.lax.broadcasted_iota(jnp.int32, (B, B), 1)
        eq = idc == idr
        last = jnp.max(jnp.where(eq, jj, -1), axis=1, keepdims=True)
        P = (jj == last).astype(jnp.float32)
        qp_ref[...] = jax.lax.dot_general(P, q, (((1,), (0,)), ((), ())),
                                          preferred_element_type=jnp.float32)

    upd = lmax > bv_scr[...]
    bv_scr[...] = jnp.where(upd, lmax, bv_scr[...])
    bi_scr[...] = jnp.where(upd, lidx, bi_scr[...])

    @pl.when(g == NT - 1)
    def _fin():
        best_ref[...] = bi_scr[...]


def _sc_wid():
    return lax.axis_index("s") * _SC_NC + lax.axis_index("c")


def _s0_body(vals_hbm, out_hbm, buf0, buf1, sem_in, sem_out):
    wid = _sc_wid()
    bufs = (buf0, buf1)

    def chunk(k):
        return wid + _S0W * k

    def in_cp(k):
        return pltpu.make_async_copy(
            vals_hbm.at[pl.ds(chunk(k) * _S0CH, _S0CH), :], bufs[k % 2], sem_in)

    def out_cp(k):
        return pltpu.make_async_copy(
            bufs[k % 2], out_hbm.at[pl.ds(chunk(k) * _S0CH, _S0CH), :], sem_out)

    for k in range(_S0K):
        if k >= 2:
            @pl.when(chunk(k - 2) < _S0NCH)
            def _(k=k):
                out_cp(k - 2).wait()

        @pl.when(chunk(k) < _S0NCH)
        def _(k=k):
            in_cp(k).start()

        if k >= 1:
            @pl.when(chunk(k - 1) < _S0NCH)
            def _(k=k):
                in_cp(k - 1).wait()
                out_cp(k - 1).start()

    @pl.when(chunk(_S0K - 1) < _S0NCH)
    def _():
        in_cp(_S0K - 1).wait()
        out_cp(_S0K - 1).start()

    if _S0K >= 2:
        @pl.when(chunk(_S0K - 2) < _S0NCH)
        def _():
            out_cp(_S0K - 2).wait()

    @pl.when(chunk(_S0K - 1) < _S0NCH)
    def _():
        out_cp(_S0K - 1).wait()


def _s1_body(best_hbm, vals_hbm, out_hbm, idx_v, rows_v, sem):
    wid = _sc_wid()

    @pl.when(wid < _NW)
    def _():
        base = wid * _ROWS_PER_W
        pltpu.sync_copy(best_hbm.at[pl.ds(base, _ROWS_PER_W)], idx_v)
        pltpu.async_copy(vals_hbm.at[idx_v], rows_v, sem).wait()
        pltpu.sync_copy(rows_v, out_hbm.at[pl.ds(base, _ROWS_PER_W)])


def _s2_body(ids_hbm, qp_hbm, cp_hbm, nk_ref, nv_ref,
             idx_v, qrows_v, crows_v, sem):
    wid = _sc_wid()

    @pl.when(wid < _NW)
    def _():
        base = wid * _ROWS_PER_W
        pltpu.sync_copy(ids_hbm.at[pl.ds(base, _ROWS_PER_W)], idx_v)
        pltpu.sync_copy(qp_hbm.at[pl.ds(base, _ROWS_PER_W)], qrows_v)
        pltpu.sync_copy(cp_hbm.at[pl.ds(base, _ROWS_PER_W)], crows_v)
        pltpu.async_copy(qrows_v, nk_ref.at[idx_v], sem).wait()
        pltpu.async_copy(crows_v, nv_ref.at[idx_v], sem).wait()


def _k2_body(best_ref, bid_ref, x_ref, h_ref, c_ref, Wi_ref, bi_ref,
             Wh_ref, bh_ref, W1_ref, b1_ref, Wa_ref, ba_ref, Wc_ref, bc_ref,
             idc_ref, idr_ref, vals_any,
             a_ref, prob_ref, v_ref, ent_ref, ht_ref, ct_ref, cp_ref,
             mem_ref, sem):
    def dotT(a, w):
        return jax.lax.dot_general(a, w, (((1,), (1,)), ((), ())),
                                   preferred_element_type=jnp.float32)

    for b in range(B):
        idx = best_ref[b]
        pltpu.make_async_copy(vals_any.at[pl.ds(idx, 1), :],
                              mem_ref.at[pl.ds(b, 1), :], sem).start()

    x = x_ref[...]
    hh = h_ref[...]
    preact = (dotT(x, Wi_ref[...]) + bi_ref[...]
              + dotT(hh, Wh_ref[...]) + bh_ref[...])
    gates = jax.nn.sigmoid(preact[:, :4 * H])
    f_t = gates[:, 0:H]
    i_t = gates[:, H:2 * H]
    o_t = gates[:, 2 * H:3 * H]
    r_t = gates[:, 3 * H:4 * H]
    c_new = jnp.tanh(preact[:, 4 * H:5 * H])
    c_t = f_t * c_ref[...] + i_t * c_new

    for b in range(B):
        idx = best_ref[b]
        pltpu.make_async_copy(vals_any.at[pl.ds(idx, 1), :],
                              mem_ref.at[pl.ds(b, 1), :], sem).wait()

    m_t = jnp.tanh(mem_ref[...])
    c_t = c_t + r_t * m_t
    h_t = o_t * jnp.tanh(c_t)
    ht_ref[...] = h_t
    ct_ref[...] = c_t

    hid = jax.nn.relu(dotT(c_t, W1_ref[...]) + b1_ref[...])
    logits = dotT(hid, Wa_ref[...]) + ba_ref[...]
    lm = jnp.max(logits, axis=1, keepdims=True)
    e = jnp.exp(logits - lm)
    pi = e / jnp.sum(e, axis=1, keepdims=True)
    ent_ref[...] = -jnp.sum(pi * jnp.log(pi + 1e-12), axis=1, keepdims=True)
    pmax = jnp.max(pi, axis=1, keepdims=True)
    ai = jax.lax.broadcasted_iota(jnp.int32, pi.shape, 1)
    a_ref[...] = jnp.min(jnp.where(pi == pmax, ai, NA), axis=1, keepdims=True)
    prob_ref[...] = jnp.log(pmax + 1e-12)
    v_ref[...] = jnp.sum(hid * Wc_ref[...], axis=1, keepdims=True) + bc_ref[0]

    idc = idc_ref[...]
    idr = idr_ref[...]
    jj = jax.lax.broadcasted_iota(jnp.int32, (B, B), 1)
    eq = idc == idr
    last = jnp.max(jnp.where(eq, jj, -1), axis=1, keepdims=True)
    P = (jj == last).astype(jnp.float32)
    cp_ref[...] = jax.lax.dot_general(P, c_t, (((1,), (0,)), ((), ())),
                                      preferred_element_type=jnp.float32)


def kernel(obs_bar_reward, barcode_tensor, barcode_id, h, c, Wi, bi, Wh, bh,
           dnd_keys, dnd_vals, W1, b1, Wa, ba, Wc, bc):
    f32 = jnp.float32
    idc = barcode_id.reshape(B, 1)
    idr = barcode_id.reshape(1, B)

    vmem = lambda shape: pl.BlockSpec(shape, lambda g: (0, 0))
    tile = lambda w: pl.BlockSpec((TILE, w), lambda g: (g, 0))

    sc_mesh = plsc.VectorSubcoreMesh(core_axis_name="c", subcore_axis_name="s")

    nv_pre = pl.kernel(
        _s0_body,
        out_type=jax.ShapeDtypeStruct((DICT, H), f32),
        mesh=sc_mesh,
        scratch_types=[pltpu.VMEM((_S0CH, H), f32),
                       pltpu.VMEM((_S0CH, H), f32),
                       pltpu.SemaphoreType.DMA,
                       pltpu.SemaphoreType.DMA],
        name="dnd_vals_copy",
        cost_estimate=pl.CostEstimate(flops=0, transcendentals=0,
                                      bytes_accessed=2 * DICT * H * 4),
    )(dnd_vals)

    nk_pre, best_id, q_payload = pl.pallas_call(
        _k1_body,
        grid=(NT,),
        in_specs=[vmem((B, KDIM)), vmem((B, 1)), vmem((1, B)),
                  tile(KDIM)],
        out_specs=[tile(KDIM), vmem((B, 1)), vmem((B, KDIM))],
        out_shape=[jax.ShapeDtypeStruct((DICT, KDIM), f32),
                   jax.ShapeDtypeStruct((B, 1), jnp.int32),
                   jax.ShapeDtypeStruct((B, KDIM), f32)],
        scratch_shapes=[pltpu.VMEM((B, 1), f32), pltpu.VMEM((B, 1), jnp.int32)],
    )(barcode_tensor, idc, idr, dnd_keys)

    smem1d = pl.BlockSpec(memory_space=pltpu.SMEM)
    vfull = pl.BlockSpec(memory_space=pltpu.VMEM)
    anyspec = pl.BlockSpec(memory_space=pl.ANY)

    (a_t, prob_a_t, v_t, entropy, h_t, c_t, c_payload) = pl.pallas_call(
        _k2_body,
        in_specs=[smem1d, smem1d, vfull, vfull, vfull, vfull, vfull,
                  vfull, vfull, vfull, vfull, vfull, vfull, vfull, smem1d,
                  vfull, vfull, anyspec],
        out_specs=[vfull, vfull, vfull, vfull, vfull, vfull, vfull],
        out_shape=[jax.ShapeDtypeStruct((B, 1), jnp.int32),
                   jax.ShapeDtypeStruct((B, 1), f32),
                   jax.ShapeDtypeStruct((B, 1), f32),
                   jax.ShapeDtypeStruct((B, 1), f32),
                   jax.ShapeDtypeStruct((B, H), f32),
                   jax.ShapeDtypeStruct((B, H), f32),
                   jax.ShapeDtypeStruct((B, H), f32)],
        scratch_shapes=[pltpu.VMEM((B, H), f32), pltpu.SemaphoreType.DMA],
    )(best_id.reshape(B), barcode_id, obs_bar_reward, h, c,
      Wi, bi.reshape(1, 5 * H), Wh, bh.reshape(1, 5 * H),
      W1, b1.reshape(1, A2CH), Wa, ba.reshape(1, NA), Wc, bc,
      idc, idr, dnd_vals)

    nk_ref = jax.new_ref(nk_pre)
    nv_ref = jax.new_ref(nv_pre)
    pl.kernel(
        _s2_body,
        out_type=(),
        mesh=sc_mesh,
        scratch_types=[pltpu.VMEM((_ROWS_PER_W,), jnp.int32),
                       pltpu.VMEM((_ROWS_PER_W, KDIM), f32),
                       pltpu.VMEM((_ROWS_PER_W, H), f32),
                       pltpu.SemaphoreType.DMA],
        name="dnd_scatter",
    )(barcode_id, q_payload, c_payload, nk_ref, nv_ref)
    new_keys = nk_ref[...]
    new_vals = nv_ref[...]

    return (a_t.reshape(B), prob_a_t.reshape(B), v_t, entropy.reshape(B),
            h_t, c_t, best_id.reshape(B), new_keys, new_vals)


# speedup vs baseline: 1.0428x; 1.0077x over previous
import functools

import jax
import jax.numpy as jnp
from jax import lax
from jax.experimental import pallas as pl
from jax.experimental.pallas import tpu as pltpu
from jax.experimental.pallas import tpu_sc as plsc

B = 64
DIN = 256
H = 128
KDIM = 128
DICT = 100000
A2CH = 128
NA = 6

TILE = 20000
NT = DICT // TILE

_SC_NC = 2
_ROWS_PER_W = 8
_NW = B // _ROWS_PER_W

_S0CH = 400
_S0NCH = DICT // _S0CH
_S0W = 32
_S0K = (_S0NCH + _S0W - 1) // _S0W


def _k1_body(q_ref, idc_ref, idr_ref, keys_ref,
             nk_ref, best_ref, qp_ref,
             bv_scr, bi_scr):
    g = pl.program_id(0)

    k_tile = keys_ref[...]
    nk_ref[...] = k_tile

    q = q_ref[...]
    qn = q / (jnp.sqrt(jnp.sum(q * q, axis=1, keepdims=True)) + 1e-8)
    kn = k_tile / (jnp.sqrt(jnp.sum(k_tile * k_tile, axis=1, keepdims=True)) + 1e-8)
    s = jax.lax.dot_general(qn, kn, (((1,), (1,)), ((), ())),
                            preferred_element_type=jnp.float32)

    lmax = jnp.max(s, axis=1, keepdims=True)
    li = jax.lax.broadcasted_iota(jnp.int32, s.shape, 1)
    lidx = jnp.min(jnp.where(s == lmax, li, TILE), axis=1, keepdims=True) + g * TILE

    @pl.when(g == 0)
    def _init():
        bv_scr[...] = jnp.full((B, 1), -jnp.inf, jnp.float32)
        bi_scr[...] = jnp.zeros((B, 1), jnp.int32)
        idc = idc_ref[...]
        idr = idr_ref[...]
        jj = jax.lax.broadcasted_iota(jnp.int32, (B, B), 1)
        eq = idc == idr
        last = jnp.max(jnp.where(eq, jj, -1), axis=1, keepdims=True)
        P = (jj == last).astype(jnp.float32)
        qp_ref[...] = jax.lax.dot_general(P, q, (((1,), (0,)), ((), ())),
                                          preferred_element_type=jnp.float32)

    upd = lmax > bv_scr[...]
    bv_scr[...] = jnp.where(upd, lmax, bv_scr[...])
    bi_scr[...] = jnp.where(upd, lidx, bi_scr[...])

    @pl.when(g == NT - 1)
    def _fin():
        best_ref[...] = bi_scr[...]


def _sc_wid():
    return lax.axis_index("s") * _SC_NC + lax.axis_index("c")


def _s0_body(vals_hbm, out_hbm, buf0, buf1, sem_in, sem_out):
    wid = _sc_wid()
    bufs = (buf0, buf1)

    def chunk(k):
        return wid + _S0W * k

    def in_cp(k):
        return pltpu.make_async_copy(
            vals_hbm.at[pl.ds(chunk(k) * _S0CH, _S0CH), :], bufs[k % 2], sem_in)

    def out_cp(k):
        return pltpu.make_async_copy(
            bufs[k % 2], out_hbm.at[pl.ds(chunk(k) * _S0CH, _S0CH), :], sem_out)

    for k in range(_S0K):
        if k >= 2:
            @pl.when(chunk(k - 2) < _S0NCH)
            def _(k=k):
                out_cp(k - 2).wait()

        @pl.when(chunk(k) < _S0NCH)
        def _(k=k):
            in_cp(k).start()

        if k >= 1:
            @pl.when(chunk(k - 1) < _S0NCH)
            def _(k=k):
                in_cp(k - 1).wait()
                out_cp(k - 1).start()

    @pl.when(chunk(_S0K - 1) < _S0NCH)
    def _():
        in_cp(_S0K - 1).wait()
        out_cp(_S0K - 1).start()

    if _S0K >= 2:
        @pl.when(chunk(_S0K - 2) < _S0NCH)
        def _():
            out_cp(_S0K - 2).wait()

    @pl.when(chunk(_S0K - 1) < _S0NCH)
    def _():
        out_cp(_S0K - 1).wait()


def _s1_body(best_hbm, vals_hbm, out_hbm, idx_v, rows_v, sem):
    wid = _sc_wid()

    @pl.when(wid < _NW)
    def _():
        base = wid * _ROWS_PER_W
        pltpu.sync_copy(best_hbm.at[pl.ds(base, _ROWS_PER_W)], idx_v)
        pltpu.async_copy(vals_hbm.at[idx_v], rows_v, sem).wait()
        pltpu.sync_copy(rows_v, out_hbm.at[pl.ds(base, _ROWS_PER_W)])


def _s2_body(ids_hbm, qp_hbm, cp_hbm, nk_ref, nv_ref,
             idx_v, qrows_v, crows_v, sem):
    wid = _sc_wid()

    @pl.when(wid < _NW)
    def _():
        base = wid * _ROWS_PER_W
        pltpu.sync_copy(ids_hbm.at[pl.ds(base, _ROWS_PER_W)], idx_v)
        pltpu.sync_copy(qp_hbm.at[pl.ds(base, _ROWS_PER_W)], qrows_v)
        pltpu.sync_copy(cp_hbm.at[pl.ds(base, _ROWS_PER_W)], crows_v)
        pltpu.async_copy(qrows_v, nk_ref.at[idx_v], sem).wait()
        pltpu.async_copy(crows_v, nv_ref.at[idx_v], sem).wait()


def _k2_body(best_ref, bid_ref, x_ref, h_ref, c_ref, Wi_ref, bi_ref,
             Wh_ref, bh_ref, W1_ref, b1_ref, Wa_ref, ba_ref, Wc_ref, bc_ref,
             idc_ref, idr_ref, vals_any,
             a_ref, prob_ref, v_ref, ent_ref, ht_ref, ct_ref, cp_ref,
             mem_ref, sem):
    def dotT(a, w):
        return jax.lax.dot_general(a, w, (((1,), (1,)), ((), ())),
                                   preferred_element_type=jnp.float32)

    for b in range(B):
        idx = best_ref[b]
        pltpu.make_async_copy(vals_any.at[pl.ds(idx, 1), :],
                              mem_ref.at[pl.ds(b, 1), :], sem).start()

    x = x_ref[...]
    hh = h_ref[...]
    preact = (dotT(x, Wi_ref[...]) + bi_ref[...]
              + dotT(hh, Wh_ref[...]) + bh_ref[...])
    gates = jax.nn.sigmoid(preact[:, :4 * H])
    f_t = gates[:, 0:H]
    i_t = gates[:, H:2 * H]
    o_t = gates[:, 2 * H:3 * H]
    r_t = gates[:, 3 * H:4 * H]
    c_new = jnp.tanh(preact[:, 4 * H:5 * H])
    c_t = f_t * c_ref[...] + i_t * c_new

    for b in range(B):
        idx = best_ref[b]
        pltpu.make_async_copy(vals_any.at[pl.ds(idx, 1), :],
                              mem_ref.at[pl.ds(b, 1), :], sem).wait()

    m_t = jnp.tanh(mem_ref[...])
    c_t = c_t + r_t * m_t
    h_t = o_t * jnp.tanh(c_t)
    ht_ref[...] = h_t
    ct_ref[...] = c_t

    hid = jax.nn.relu(dotT(c_t, W1_ref[...]) + b1_ref[...])
    logits = dotT(hid, Wa_ref[...]) + ba_ref[...]
    lm = jnp.max(logits, axis=1, keepdims=True)
    e = jnp.exp(logits - lm)
    pi = e / jnp.sum(e, axis=1, keepdims=True)
    ent_ref[...] = -jnp.sum(pi * jnp.log(pi + 1e-12), axis=1, keepdims=True)
    pmax = jnp.max(pi, axis=1, keepdims=True)
    ai = jax.lax.broadcasted_iota(jnp.int32, pi.shape, 1)
    a_ref[...] = jnp.min(jnp.where(pi == pmax, ai, NA), axis=1, keepdims=True)
    prob_ref[...] = jnp.log(pmax + 1e-12)
    v_ref[...] = jnp.sum(hid * Wc_ref[...], axis=1, keepdims=True) + bc_ref[0]

    idc = idc_ref[...]
    idr = idr_ref[...]
    jj = jax.lax.broadcasted_iota(jnp.int32, (B, B), 1)
    eq = idc == idr
    last = jnp.max(jnp.where(eq, jj, -1), axis=1, keepdims=True)
    P = (jj == last).astype(jnp.float32)
    cp_ref[...] = jax.lax.dot_general(P, c_t, (((1,), (0,)), ((), ())),
                                      preferred_element_type=jnp.float32)


def kernel(obs_bar_reward, barcode_tensor, barcode_id, h, c, Wi, bi, Wh, bh,
           dnd_keys, dnd_vals, W1, b1, Wa, ba, Wc, bc):
    f32 = jnp.float32
    idc = barcode_id.reshape(B, 1)
    idr = barcode_id.reshape(1, B)

    vmem = lambda shape: pl.BlockSpec(shape, lambda g: (0, 0))
    tile = lambda w: pl.BlockSpec((TILE, w), lambda g: (g, 0))

    sc_mesh = plsc.VectorSubcoreMesh(core_axis_name="c", subcore_axis_name="s")

    nv_pre = pl.kernel(
        _s0_body,
        out_type=jax.ShapeDtypeStruct((DICT, H), f32),
        mesh=sc_mesh,
        scratch_types=[pltpu.VMEM((_S0CH, H), f32),
                       pltpu.VMEM((_S0CH, H), f32),
                       pltpu.SemaphoreType.DMA,
                       pltpu.SemaphoreType.DMA],
        name="dnd_vals_copy",
        cost_estimate=pl.CostEstimate(flops=0, transcendentals=0,
                                      bytes_accessed=2 * DICT * H * 4),
    )(dnd_vals)

    nk_pre, best_id, q_payload = pl.pallas_call(
        _k1_body,
        grid=(NT,),
        in_specs=[vmem((B, KDIM)), vmem((B, 1)), vmem((1, B)),
                  tile(KDIM)],
        out_specs=[tile(KDIM), vmem((B, 1)), vmem((B, KDIM))],
        out_shape=[jax.ShapeDtypeStruct((DICT, KDIM), f32),
                   jax.ShapeDtypeStruct((B, 1), jnp.int32),
                   jax.ShapeDtypeStruct((B, KDIM), f32)],
        scratch_shapes=[pltpu.VMEM((B, 1), f32), pltpu.VMEM((B, 1), jnp.int32)],
    )(barcode_tensor, idc, idr, dnd_keys)

    smem1d = pl.BlockSpec(memory_space=pltpu.SMEM)
    vfull = pl.BlockSpec(memory_space=pltpu.VMEM)
    anyspec = pl.BlockSpec(memory_space=pl.ANY)

    (a_t, prob_a_t, v_t, entropy, h_t, c_t, c_payload) = pl.pallas_call(
        _k2_body,
        in_specs=[smem1d, smem1d, vfull, vfull, vfull, vfull, vfull,
                  vfull, vfull, vfull, vfull, vfull, vfull, vfull, smem1d,
                  vfull, vfull, anyspec],
        out_specs=[vfull, vfull, vfull, vfull, vfull, vfull, vfull],
        out_shape=[jax.ShapeDtypeStruct((B, 1), jnp.int32),
                   jax.ShapeDtypeStruct((B, 1), f32),
                   jax.ShapeDtypeStruct((B, 1), f32),
                   jax.ShapeDtypeStruct((B, 1), f32),
                   jax.ShapeDtypeStruct((B, H), f32),
                   jax.ShapeDtypeStruct((B, H), f32),
                   jax.ShapeDtypeStruct((B, H), f32)],
        scratch_shapes=[pltpu.VMEM((B, H), f32), pltpu.SemaphoreType.DMA],
    )(best_id.reshape(B), barcode_id, obs_bar_reward, h, c,
      Wi, bi.reshape(1, 5 * H), Wh, bh.reshape(1, 5 * H),
      W1, b1.reshape(1, A2CH), Wa, ba.reshape(1, NA), Wc, bc,
      idc, idr, dnd_vals)

    nk_ref = jax.new_ref(nk_pre)
    nv_ref = jax.new_ref(nv_pre)
    pl.kernel(
        _s2_body,
        out_type=(),
        mesh=sc_mesh,
        scratch_types=[pltpu.VMEM((_ROWS_PER_W,), jnp.int32),
                       pltpu.VMEM((_ROWS_PER_W, KDIM), f32),
                       pltpu.VMEM((_ROWS_PER_W, H), f32),
                       pltpu.SemaphoreType.DMA],
        name="dnd_scatter",
    )(barcode_id, q_payload, c_payload, nk_ref, nv_ref)
    new_keys = nk_ref[...]
    new_vals = nv_ref[...]

    return (a_t.reshape(B), prob_a_t.reshape(B), v_t, entropy.reshape(B),
            h_t, c_t, best_id.reshape(B), new_keys, new_vals)
